# baseline (speedup 1.0000x reference)
import functools

import jax
import jax.numpy as jnp
from jax import lax
from jax.experimental import pallas as pl
from jax.experimental.pallas import tpu as pltpu
from jax.experimental.pallas import tpu_sc as plsc

N = 10000
NP = 10240
E = 160000
C = 128
NC = 2
NS = 16
EPT = E // NS
ECH = 50
NECH = EPT // ECH
W = 8
NW = NECH // W
NQ = NECH // 4
RPT = NP // NS
RCH = 8
NRCH = RPT // RCH
POW = (1, 2, 4, 8, 16)
STEPS = 16

_mesh = plsc.VectorSubcoreMesh(
    core_axis_name="c", subcore_axis_name="s", num_cores=NC, num_subcores=NS
)


def _fill(ref, rows, val):
    def body(i, carry):
        r = i // (C // 16)
        g = (i % (C // 16)) * 16
        ref[r, pl.ds(g, 16)] = jnp.full((16,), val, jnp.float32)
        return carry
    lax.fori_loop(0, rows * (C // 16), body, 0)


def _prep_body(dst3, zeros_in, cexp_out, deg_sp, dstb, ones, dbuf, cbuf):
    c = lax.axis_index("c")
    s = lax.axis_index("s")

    @pl.when(c == 0)
    def _():
        base = s * RPT
        _fill(ones, ECH, 1.0)
        for k in range(0, NRCH, 4):
            pltpu.sync_copy(zeros_in, deg_sp.at[pl.ds(base + k * RCH, RCH)])
            pltpu.sync_copy(
                zeros_in, deg_sp.at[pl.ds(base + (k + 1) * RCH, RCH)])
            pltpu.sync_copy(
                zeros_in, deg_sp.at[pl.ds(base + (k + 2) * RCH, RCH)])
            pltpu.sync_copy(
                zeros_in, deg_sp.at[pl.ds(base + (k + 3) * RCH, RCH)])
        pltpu.sync_copy(dst3.at[s], dstb)
        plsc.subcore_barrier()

        def ebody(j, carry):
            pltpu.sync_copy(ones, deg_sp.at[dstb.at[j]], add=True)
            return carry
        lax.fori_loop(0, NECH, ebody, 0)
        plsc.subcore_barrier()

        for k in range(0, NRCH, 4):
            pltpu.sync_copy(
                deg_sp.at[pl.ds(base + k * RCH, 4 * RCH)], dbuf)

            def cbody(i, carry):
                r = i // (C // 16)
                g = (i % (C // 16)) * 16
                dv = dbuf[r, pl.ds(g, 16)]
                cbuf[r, pl.ds(g, 16)] = jnp.where(dv > 0.0, 0.5 / dv, 0.0)
                return carry
            lax.fori_loop(0, 4 * RCH * (C // 16), cbody, 0)
            pltpu.sync_copy(cbuf, cexp_out.at[pl.ds(base + k * RCH, 4 * RCH)])


_prep = functools.partial(
    pl.kernel,
    _prep_body,
    out_type=jax.ShapeDtypeStruct((NP, C), jnp.float32),
    mesh=_mesh,
    scratch_types=[
        pltpu.VMEM_SHARED((NP, C), jnp.float32),
        pltpu.VMEM((NECH, ECH), jnp.int32),
        pltpu.VMEM((ECH, C), jnp.float32),
        pltpu.VMEM((4 * RCH, C), jnp.float32),
        pltpu.VMEM((4 * RCH, C), jnp.float32),
    ],
)()


def _diffuse(src3, dst3, cexp, seed, zeros_in, hist,
             agg_sp, srcw, dstw, rows, ubufs,
             gsems, ssems, semw, usems, wsems, s):
    base = s * RPT
    (hbA, abA, cbA, hbB, abB, cbB) = ubufs
    usemA, usemB = usems
    wsemA, wsemB = wsems

    for k in range(0, NRCH, 2):
        off = base + k * RCH
        pltpu.sync_copy(seed.at[pl.ds(off, RCH)], hbA)
        pltpu.sync_copy(seed.at[pl.ds(off + RCH, RCH)], hbB)
        pltpu.sync_copy(hbA, hist.at[0, pl.ds(off, RCH)])
        pltpu.sync_copy(hbB, hist.at[0, pl.ds(off + RCH, RCH)])
        pltpu.sync_copy(zeros_in, agg_sp.at[pl.ds(off, RCH)])
        pltpu.sync_copy(zeros_in, agg_sp.at[pl.ds(off + RCH, RCH)])
    plsc.subcore_barrier()

    def winload(wi, slot, sync):
        sc = pltpu.sync_copy if sync else (
            lambda a, b: pltpu.async_copy(a, b, semw))
        sc(src3.at[s, pl.ds(wi * W, W)], srcw.at[slot])
        sc(dst3.at[s, pl.ds(wi * W, W)], dstw.at[slot])

    def winwait():
        pltpu.make_async_copy(
            src3.at[s, pl.ds(0, W)], srcw.at[0], semw).wait()
        pltpu.make_async_copy(
            dst3.at[s, pl.ds(0, W)], dstw.at[0], semw).wait()

    def step(t, carry):
        hprev = hist.at[t - 1]
        winload(0, 0, True)
        winload(1, 1, False)

        def fire(j, rbuf, sem):
            wi = j // W
            wl = j - wi * W
            pltpu.async_copy(hprev.at[srcw.at[wi % 2, wl]], rbuf, sem)

        def gwait(j, rbuf, sem):
            wi = j // W
            wl = j - wi * W
            pltpu.make_async_copy(
                hprev.at[srcw.at[wi % 2, wl]], rbuf, sem).wait()

        def sfire(j, rbuf, ssem):
            wi = j // W
            wl = j - wi * W
            return pltpu.async_copy(
                rbuf, agg_sp.at[dstw.at[wi % 2, wl]], ssem, add=True)

        def winmgmt(j):
            wi = j // W
            wl = j - wi * W
            @pl.when((wl == 0) & (wi >= 1) & (wi < NW - 1))
            def _():
                winload(wi + 1, (wi + 1) % 2, False)

            @pl.when((wl == 4) & (wi < NW - 1))
            def _():
                winwait()

        for u in range(4):
            fire(u, rows[u], gsems[u])

        def quad(q, icarry):
            j0 = q * 4
            cps = []
            for u in range(4):
                gwait(j0 + u, rows[u], gsems[u])
                cps.append(sfire(j0 + u, rows[u], ssems[u]))
            for u in range(4):
                winmgmt(j0 + u)
            for u in range(4):
                cps[u].wait()
                fire(j0 + u + 4, rows[u], gsems[u])
            return icarry
        lax.fori_loop(0, NQ - 1, quad, 0)
        jt = (NQ - 1) * 4
        tail_cps = []
        for u in range(4):
            gwait(jt + u, rows[u], gsems[u])
            tail_cps.append(sfire(jt + u, rows[u], ssems[u]))
        for u in range(4):
            tail_cps[u].wait()
        plsc.subcore_barrier()

        def uload(k, hb, ab, cb, usem):
            off = base + k * RCH
            pltpu.async_copy(agg_sp.at[pl.ds(off, RCH)], ab, usem)
            pltpu.async_copy(hist.at[t - 1, pl.ds(off, RCH)], hb, usem)
            pltpu.async_copy(cexp.at[pl.ds(off, RCH)], cb, usem)

        def uwait(k, hb, ab, cb, usem):
            off = base + k * RCH
            pltpu.make_async_copy(
                agg_sp.at[pl.ds(off, RCH)], ab, usem).wait()
            pltpu.make_async_copy(
                hist.at[t - 1, pl.ds(off, RCH)], hb, usem).wait()
            pltpu.make_async_copy(
                cexp.at[pl.ds(off, RCH)], cb, usem).wait()

        def ucompute(hb, ab, cb):
            def ubody(i, icarry):
                r = i // (C // 16)
                g = (i % (C // 16)) * 16
                hv = hb[r, pl.ds(g, 16)]
                av = ab[r, pl.ds(g, 16)]
                cv = cb[r, pl.ds(g, 16)]
                hb[r, pl.ds(g, 16)] = 0.5 * hv + cv * av
                return icarry
            lax.fori_loop(0, RCH * (C // 16), ubody, 0)

        def ustore(k, hb, wsem):
            off = base + k * RCH
            pltpu.async_copy(hb, hist.at[t, pl.ds(off, RCH)], wsem)
            pltpu.async_copy(zeros_in, agg_sp.at[pl.ds(off, RCH)], wsem)

        def uswait(k, hb, wsem):
            off = base + k * RCH
            pltpu.make_async_copy(
                hb, hist.at[t, pl.ds(off, RCH)], wsem).wait()
            pltpu.make_async_copy(
                zeros_in, agg_sp.at[pl.ds(off, RCH)], wsem).wait()

        def uchunk(k, icarry):
            off = base + k * RCH
            pltpu.sync_copy(agg_sp.at[pl.ds(off, RCH)], abA)
            pltpu.sync_copy(hist.at[t - 1, pl.ds(off, RCH)], hbA)
            pltpu.sync_copy(cexp.at[pl.ds(off, RCH)], cbA)
            ucompute(hbA, abA, cbA)
            pltpu.sync_copy(hbA, hist.at[t, pl.ds(off, RCH)])
            pltpu.sync_copy(zeros_in, agg_sp.at[pl.ds(off, RCH)])
            return icarry
        lax.fori_loop(0, NRCH, uchunk, 0)
        plsc.subcore_barrier()
        return carry
    lax.fori_loop(1, STEPS + 1, step, 0)


def _unpack_scratch(scr):
    agg_sp = scr[0]
    srcw, dstw = scr[1], scr[2]
    rows = scr[3:7]
    ubufs = scr[7:13]
    gsems = scr[13:17]
    ssems = scr[17:21]
    semw = scr[21]
    usems = scr[22:24]
    wsems = scr[24:26]
    return agg_sp, srcw, dstw, rows, ubufs, gsems, ssems, semw, usems, wsems


def _run_body(src3, dst3, cexp, seed, zeros_in, hist, *scr):
    c = lax.axis_index("c")
    s = lax.axis_index("s")

    @pl.when(c == 0)
    def _():
        _diffuse(src3, dst3, cexp, seed, zeros_in, hist,
                 *_unpack_scratch(scr), s)


def _run2_body(src3, dst3, cexp, seed_a, seed_b, zeros_in, hist_a, hist_b,
               *scr):
    c = lax.axis_index("c")
    s = lax.axis_index("s")

    @pl.when(c == 0)
    def _():
        _diffuse(src3, dst3, cexp, seed_a, zeros_in, hist_a,
                 *_unpack_scratch(scr), s)

    @pl.when(c == 1)
    def _():
        _diffuse(src3, dst3, cexp, seed_b, zeros_in, hist_b,
                 *_unpack_scratch(scr), s)


_SC_SCRATCH = (
    [pltpu.VMEM_SHARED((NP, C), jnp.float32)]
    + [pltpu.VMEM((2, W, ECH), jnp.int32)] * 2
    + [pltpu.VMEM((ECH, C), jnp.float32)] * 4
    + [pltpu.VMEM((RCH, C), jnp.float32)] * 6
    + [pltpu.SemaphoreType.DMA] * 13
)

_run2 = functools.partial(
    pl.kernel,
    _run2_body,
    out_type=(jax.ShapeDtypeStruct((STEPS + 1, NP, C), jnp.float32),
              jax.ShapeDtypeStruct((STEPS + 1, NP, C), jnp.float32)),
    mesh=_mesh,
    scratch_types=_SC_SCRATCH,
)()

_run = functools.partial(
    pl.kernel,
    _run_body,
    out_type=jax.ShapeDtypeStruct((STEPS + 1, NP, C), jnp.float32),
    mesh=_mesh,
    scratch_types=_SC_SCRATCH,
)()


_SEED_BN = 1024


def _seed_body(h1, h2, h4, h8, wc, o0, o1, o2):
    hs = {1: h1, 2: h2, 4: h4, 8: h8}
    for j, out in enumerate((o0, o1, o2)):
        a, b = POW[j], POW[j + 1]
        out[...] = jnp.abs(hs[a][0] * wc[j:j + 1, 0:1]
                           + hs[b][0] * wc[j:j + 1, 1:2])


def _seed_call(hist1, wcoef):
    bspec = [
        pl.BlockSpec((1, _SEED_BN, C), lambda ii, t=t: (t, ii, 0))
        for t in (1, 2, 4, 8)
    ]
    bspec.append(pl.BlockSpec((8, 128), lambda ii: (0, 0)))
    out_spec = pl.BlockSpec((_SEED_BN, C), lambda ii: (ii, 0))
    return pl.pallas_call(
        _seed_body,
        grid=(NP // _SEED_BN,),
        in_specs=bspec,
        out_specs=[out_spec] * 3,
        out_shape=[jax.ShapeDtypeStruct((NP, C), jnp.float32)] * 3,
    )(hist1, hist1, hist1, hist1, wcoef)


_FIN_BN = 400
_H1_SLOTS = (1, 2, 4, 8, 16)
_HB_SLOTS = ((2, 4, 8, 16), (4, 8, 16), (8, 16))


def _fin_body(*refs):
    x_r = refs[0]
    nh1 = len(_H1_SLOTS)
    h1refs = refs[1:1 + nh1]
    pos = 1 + nh1
    hbrefs = []
    for slots in _HB_SLOTS:
        hbrefs.append(refs[pos:pos + len(slots)])
        pos += len(slots)
    wc, Wm, bb, out = refs[pos], refs[pos + 1], refs[pos + 2], refs[pos + 3]

    h1 = {t: r[0] for t, r in zip(_H1_SLOTS, h1refs)}
    hb = [{t: r[0] for t, r in zip(slots, rs)}
          for slots, rs in zip(_HB_SLOTS, hbrefs)]

    def wav(j, ha, hbv):
        return jnp.abs(ha * wc[j:j + 1, 0:1] + hbv * wc[j:j + 1, 1:2])

    feats = [x_r[...]]
    for j in range(4):
        feats.append(wav(j, h1[POW[j]], h1[POW[j + 1]]))
    for j in range(3):
        for jp in range(j + 1, 4):
            feats.append(wav(jp, hb[j][POW[jp]], hb[j][POW[jp + 1]]))
    f = jnp.concatenate(feats, axis=-1)
    f = jnp.where(f >= 0.0, f, 0.01 * f)
    acc = lax.dot_general(f, Wm[...], (((1,), (1,)), ((), ())),
                          preferred_element_type=jnp.float32)
    out[...] = acc + bb[...]


def _fin_call(x, hist1, hbs, wcoef, lin_W, lin_b2):
    in_specs = [pl.BlockSpec((_FIN_BN, C), lambda i: (i, 0))]
    args = [x]
    for t in _H1_SLOTS:
        in_specs.append(
            pl.BlockSpec((1, _FIN_BN, C), lambda i, t=t: (t, i, 0)))
        args.append(hist1)
    for bi, slots in enumerate(_HB_SLOTS):
        for t in slots:
            in_specs.append(
                pl.BlockSpec((1, _FIN_BN, C), lambda i, t=t: (t, i, 0)))
            args.append(hbs[bi])
    in_specs.append(pl.BlockSpec((8, 128), lambda i: (0, 0)))
    args.append(wcoef)
    in_specs.append(pl.BlockSpec((C, 11 * C), lambda i: (0, 0)))
    args.append(lin_W)
    in_specs.append(pl.BlockSpec((1, C), lambda i: (0, 0)))
    args.append(lin_b2)
    return pl.pallas_call(
        _fin_body,
        grid=(N // _FIN_BN,),
        in_specs=in_specs,
        out_specs=pl.BlockSpec((_FIN_BN, C), lambda i: (i, 0)),
        out_shape=jax.ShapeDtypeStruct((N, C), jnp.float32),
    )(*args)


def kernel(x, edge_index, wavelet, lin_W, lin_b):
    src3 = edge_index[0].reshape(NS, NECH, ECH)
    dst3 = edge_index[1].reshape(NS, NECH, ECH)
    xp = jnp.zeros((NP, C), jnp.float32).at[:N].set(x)
    zeros_in = jnp.zeros((RCH, C), jnp.float32)
    wcoef = jnp.zeros((8, 128), jnp.float32)
    for j in range(4):
        wcoef = wcoef.at[j, 0].set(wavelet[j, POW[j]])
        wcoef = wcoef.at[j, 1].set(wavelet[j, POW[j + 1]])
    cexp = _prep(dst3, zeros_in)
    hist1 = _run(src3, dst3, cexp, xp, zeros_in)
    seeds = _seed_call(hist1, wcoef)
    hb0, hb1 = _run2(src3, dst3, cexp, seeds[0], seeds[1], zeros_in)
    hb2 = _run(src3, dst3, cexp, seeds[2], zeros_in)
    out = _fin_call(x, hist1, [hb0, hb1, hb2], wcoef, lin_W,
                    lin_b.reshape(1, C))
    return out, wavelet


# speedup vs baseline: 1.3367x; 1.3367x over previous
import functools

import jax
import jax.numpy as jnp
from jax import lax
from jax.experimental import pallas as pl
from jax.experimental.pallas import tpu as pltpu
from jax.experimental.pallas import tpu_sc as plsc

N = 10000
NP = 10240
E = 160000
C = 128
NC = 2
NS = 16
EPT = E // NS
ECH = 50
NECH = EPT // ECH
W = 8
NW = NECH // W
NQ = NECH // 4
RPT = NP // NS
RCH = 16
NRCH = RPT // RCH
POW = (1, 2, 4, 8, 16)
STEPS = 16

_mesh = plsc.VectorSubcoreMesh(
    core_axis_name="c", subcore_axis_name="s", num_cores=NC, num_subcores=NS
)


def _fill(ref, rows, val):
    def body(i, carry):
        r = i // (C // 16)
        g = (i % (C // 16)) * 16
        ref[r, pl.ds(g, 16)] = jnp.full((16,), val, jnp.float32)
        return carry
    lax.fori_loop(0, rows * (C // 16), body, 0)


def _prep_body(dst3, zeros_in, cexp_out, deg_sp, dstb, ones, dbuf, cbuf):
    c = lax.axis_index("c")
    s = lax.axis_index("s")

    @pl.when(c == 0)
    def _():
        base = s * RPT
        _fill(ones, ECH, 1.0)
        for k in range(0, NRCH, 4):
            pltpu.sync_copy(zeros_in, deg_sp.at[pl.ds(base + k * RCH, RCH)])
            pltpu.sync_copy(
                zeros_in, deg_sp.at[pl.ds(base + (k + 1) * RCH, RCH)])
            pltpu.sync_copy(
                zeros_in, deg_sp.at[pl.ds(base + (k + 2) * RCH, RCH)])
            pltpu.sync_copy(
                zeros_in, deg_sp.at[pl.ds(base + (k + 3) * RCH, RCH)])
        pltpu.sync_copy(dst3.at[s], dstb)
        plsc.subcore_barrier()

        def ebody(j, carry):
            pltpu.sync_copy(ones, deg_sp.at[dstb.at[j]], add=True)
            return carry
        lax.fori_loop(0, NECH, ebody, 0)
        plsc.subcore_barrier()

        for k in range(0, NRCH, 4):
            pltpu.sync_copy(
                deg_sp.at[pl.ds(base + k * RCH, 4 * RCH)], dbuf)

            def cbody(i, carry):
                r = i // (C // 16)
                g = (i % (C // 16)) * 16
                dv = dbuf[r, pl.ds(g, 16)]
                cbuf[r, pl.ds(g, 16)] = jnp.where(dv > 0.0, 0.5 / dv, 0.0)
                return carry
            lax.fori_loop(0, 4 * RCH * (C // 16), cbody, 0)
            pltpu.sync_copy(cbuf, cexp_out.at[pl.ds(base + k * RCH, 4 * RCH)])


_prep = functools.partial(
    pl.kernel,
    _prep_body,
    out_type=jax.ShapeDtypeStruct((NP, C), jnp.float32),
    mesh=_mesh,
    scratch_types=[
        pltpu.VMEM_SHARED((NP, C), jnp.float32),
        pltpu.VMEM((NECH, ECH), jnp.int32),
        pltpu.VMEM((ECH, C), jnp.float32),
        pltpu.VMEM((4 * RCH, C), jnp.float32),
        pltpu.VMEM((4 * RCH, C), jnp.float32),
    ],
)()


def _diffuse(src3, dst3, cexp, seed, zeros_in, hist,
             agg_sp, srcw, dstw, rows, ubufs,
             gsems, ssems, semw, s):
    base = s * RPT
    (hbA, abA, cbA) = ubufs

    for k in range(NRCH):
        off = base + k * RCH
        pltpu.sync_copy(seed.at[pl.ds(off, RCH)], hbA)
        pltpu.sync_copy(hbA, hist.at[0, pl.ds(off, RCH)])
        pltpu.sync_copy(zeros_in, agg_sp.at[pl.ds(off, RCH)])
    plsc.subcore_barrier()

    def winload(wi, slot, sync):
        sc = pltpu.sync_copy if sync else (
            lambda a, b: pltpu.async_copy(a, b, semw))
        sc(src3.at[s, pl.ds(wi * W, W)], srcw.at[slot])
        sc(dst3.at[s, pl.ds(wi * W, W)], dstw.at[slot])

    def winwait():
        pltpu.make_async_copy(
            src3.at[s, pl.ds(0, W)], srcw.at[0], semw).wait()
        pltpu.make_async_copy(
            dst3.at[s, pl.ds(0, W)], dstw.at[0], semw).wait()

    def step(t, carry):
        hprev = hist.at[t - 1]
        winload(0, 0, True)
        winload(1, 1, False)

        def fire(j, rbuf, sem):
            wi = j // W
            wl = j - wi * W
            pltpu.async_copy(hprev.at[srcw.at[wi % 2, wl]], rbuf, sem)

        def gwait(j, rbuf, sem):
            wi = j // W
            wl = j - wi * W
            pltpu.make_async_copy(
                hprev.at[srcw.at[wi % 2, wl]], rbuf, sem).wait()

        def sfire(j, rbuf, ssem):
            wi = j // W
            wl = j - wi * W
            return pltpu.async_copy(
                rbuf, agg_sp.at[dstw.at[wi % 2, wl]], ssem, add=True)

        def winmgmt(j):
            wi = j // W
            wl = j - wi * W
            @pl.when((wl == 0) & (wi >= 1) & (wi < NW - 1))
            def _():
                winload(wi + 1, (wi + 1) % 2, False)

            @pl.when((wl == 4) & (wi < NW - 1))
            def _():
                winwait()

        for u in range(4):
            fire(u, rows[u], gsems[u])

        def quad(q, icarry):
            j0 = q * 4
            cps = []
            for u in range(4):
                gwait(j0 + u, rows[u], gsems[u])
                cps.append(sfire(j0 + u, rows[u], ssems[u]))
            for u in range(4):
                winmgmt(j0 + u)
            for u in range(4):
                cps[u].wait()
                fire(j0 + u + 4, rows[u], gsems[u])
            return icarry
        lax.fori_loop(0, NQ - 1, quad, 0)
        jt = (NQ - 1) * 4
        tail_cps = []
        for u in range(4):
            gwait(jt + u, rows[u], gsems[u])
            tail_cps.append(sfire(jt + u, rows[u], ssems[u]))
        for u in range(4):
            tail_cps[u].wait()
        plsc.subcore_barrier()

        def ucompute(hb, ab, cb):
            def ubody(i, icarry):
                r = i // (C // 16)
                g = (i % (C // 16)) * 16
                hv = hb[r, pl.ds(g, 16)]
                av = ab[r, pl.ds(g, 16)]
                cv = cb[r, pl.ds(g, 16)]
                hb[r, pl.ds(g, 16)] = 0.5 * hv + cv * av
                return icarry
            lax.fori_loop(0, RCH * (C // 16), ubody, 0)

        def ustore(k, hb, wsem):
            off = base + k * RCH
            pltpu.async_copy(hb, hist.at[t, pl.ds(off, RCH)], wsem)
            pltpu.async_copy(zeros_in, agg_sp.at[pl.ds(off, RCH)], wsem)

        def uswait(k, hb, wsem):
            off = base + k * RCH
            pltpu.make_async_copy(
                hb, hist.at[t, pl.ds(off, RCH)], wsem).wait()
            pltpu.make_async_copy(
                zeros_in, agg_sp.at[pl.ds(off, RCH)], wsem).wait()

        def uchunk(k, icarry):
            off = base + k * RCH
            pltpu.sync_copy(agg_sp.at[pl.ds(off, RCH)], abA)
            pltpu.sync_copy(hist.at[t - 1, pl.ds(off, RCH)], hbA)
            pltpu.sync_copy(cexp.at[pl.ds(off, RCH)], cbA)
            ucompute(hbA, abA, cbA)
            pltpu.sync_copy(hbA, hist.at[t, pl.ds(off, RCH)])
            pltpu.sync_copy(zeros_in, agg_sp.at[pl.ds(off, RCH)])
            return icarry
        lax.fori_loop(0, NRCH, uchunk, 0)
        plsc.subcore_barrier()
        return carry
    lax.fori_loop(1, STEPS + 1, step, 0)


def _unpack_scratch(scr):
    agg_sp = scr[0]
    srcw, dstw = scr[1], scr[2]
    rows = scr[3:7]
    ubufs = scr[7:10]
    gsems = scr[10:14]
    ssems = scr[14:18]
    semw = scr[18]
    return agg_sp, srcw, dstw, rows, ubufs, gsems, ssems, semw


def _run_body(src3, dst3, cexp, seed, zeros_in, hist, *scr):
    c = lax.axis_index("c")
    s = lax.axis_index("s")

    @pl.when(c == 0)
    def _():
        _diffuse(src3, dst3, cexp, seed, zeros_in, hist,
                 *_unpack_scratch(scr), s)


def _run2_body(src3, dst3, cexp, seed_a, seed_b, zeros_in, hist_a, hist_b,
               *scr):
    c = lax.axis_index("c")
    s = lax.axis_index("s")

    @pl.when(c == 0)
    def _():
        _diffuse(src3, dst3, cexp, seed_a, zeros_in, hist_a,
                 *_unpack_scratch(scr), s)

    @pl.when(c == 1)
    def _():
        _diffuse(src3, dst3, cexp, seed_b, zeros_in, hist_b,
                 *_unpack_scratch(scr), s)


_SC_SCRATCH = (
    [pltpu.VMEM_SHARED((NP, C), jnp.float32)]
    + [pltpu.VMEM((2, W, ECH), jnp.int32)] * 2
    + [pltpu.VMEM((ECH, C), jnp.float32)] * 4
    + [pltpu.VMEM((RCH, C), jnp.float32)] * 3
    + [pltpu.SemaphoreType.DMA] * 9
)

_run2 = functools.partial(
    pl.kernel,
    _run2_body,
    out_type=(jax.ShapeDtypeStruct((STEPS + 1, NP, C), jnp.float32),
              jax.ShapeDtypeStruct((STEPS + 1, NP, C), jnp.float32)),
    mesh=_mesh,
    scratch_types=_SC_SCRATCH,
)()

_run = functools.partial(
    pl.kernel,
    _run_body,
    out_type=jax.ShapeDtypeStruct((STEPS + 1, NP, C), jnp.float32),
    mesh=_mesh,
    scratch_types=_SC_SCRATCH,
)()


_SEED_BN = 1024


def _seed_body(h1, h2, h4, h8, wc, o0, o1, o2):
    hs = {1: h1, 2: h2, 4: h4, 8: h8}
    for j, out in enumerate((o0, o1, o2)):
        a, b = POW[j], POW[j + 1]
        out[...] = jnp.abs(hs[a][0] * wc[j:j + 1, 0:1]
                           + hs[b][0] * wc[j:j + 1, 1:2])


def _seed_call(hist1, wcoef):
    bspec = [
        pl.BlockSpec((1, _SEED_BN, C), lambda ii, t=t: (t, ii, 0))
        for t in (1, 2, 4, 8)
    ]
    bspec.append(pl.BlockSpec((8, 128), lambda ii: (0, 0)))
    out_spec = pl.BlockSpec((_SEED_BN, C), lambda ii: (ii, 0))
    return pl.pallas_call(
        _seed_body,
        grid=(NP // _SEED_BN,),
        in_specs=bspec,
        out_specs=[out_spec] * 3,
        out_shape=[jax.ShapeDtypeStruct((NP, C), jnp.float32)] * 3,
    )(hist1, hist1, hist1, hist1, wcoef)


_FIN_BN = 400
_H1_SLOTS = (1, 2, 4, 8, 16)
_HB_SLOTS = ((2, 4, 8, 16), (4, 8, 16), (8, 16))


def _fin_body(*refs):
    x_r = refs[0]
    nh1 = len(_H1_SLOTS)
    h1refs = refs[1:1 + nh1]
    pos = 1 + nh1
    hbrefs = []
    for slots in _HB_SLOTS:
        hbrefs.append(refs[pos:pos + len(slots)])
        pos += len(slots)
    wc, Wm, bb, out = refs[pos], refs[pos + 1], refs[pos + 2], refs[pos + 3]

    h1 = {t: r[0] for t, r in zip(_H1_SLOTS, h1refs)}
    hb = [{t: r[0] for t, r in zip(slots, rs)}
          for slots, rs in zip(_HB_SLOTS, hbrefs)]

    def wav(j, ha, hbv):
        return jnp.abs(ha * wc[j:j + 1, 0:1] + hbv * wc[j:j + 1, 1:2])

    feats = [x_r[...]]
    for j in range(4):
        feats.append(wav(j, h1[POW[j]], h1[POW[j + 1]]))
    for j in range(3):
        for jp in range(j + 1, 4):
            feats.append(wav(jp, hb[j][POW[jp]], hb[j][POW[jp + 1]]))
    f = jnp.concatenate(feats, axis=-1)
    f = jnp.where(f >= 0.0, f, 0.01 * f)
    acc = lax.dot_general(f, Wm[...], (((1,), (1,)), ((), ())),
                          preferred_element_type=jnp.float32)
    out[...] = acc + bb[...]


def _fin_call(x, hist1, hbs, wcoef, lin_W, lin_b2):
    in_specs = [pl.BlockSpec((_FIN_BN, C), lambda i: (i, 0))]
    args = [x]
    for t in _H1_SLOTS:
        in_specs.append(
            pl.BlockSpec((1, _FIN_BN, C), lambda i, t=t: (t, i, 0)))
        args.append(hist1)
    for bi, slots in enumerate(_HB_SLOTS):
        for t in slots:
            in_specs.append(
                pl.BlockSpec((1, _FIN_BN, C), lambda i, t=t: (t, i, 0)))
            args.append(hbs[bi])
    in_specs.append(pl.BlockSpec((8, 128), lambda i: (0, 0)))
    args.append(wcoef)
    in_specs.append(pl.BlockSpec((C, 11 * C), lambda i: (0, 0)))
    args.append(lin_W)
    in_specs.append(pl.BlockSpec((1, C), lambda i: (0, 0)))
    args.append(lin_b2)
    return pl.pallas_call(
        _fin_body,
        grid=(N // _FIN_BN,),
        in_specs=in_specs,
        out_specs=pl.BlockSpec((_FIN_BN, C), lambda i: (i, 0)),
        out_shape=jax.ShapeDtypeStruct((N, C), jnp.float32),
    )(*args)


def kernel(x, edge_index, wavelet, lin_W, lin_b):
    src3 = edge_index[0].reshape(NS, NECH, ECH)
    dst3 = edge_index[1].reshape(NS, NECH, ECH)
    xp = jnp.zeros((NP, C), jnp.float32).at[:N].set(x)
    zeros_in = jnp.zeros((RCH, C), jnp.float32)
    wcoef = jnp.zeros((8, 128), jnp.float32)
    for j in range(4):
        wcoef = wcoef.at[j, 0].set(wavelet[j, POW[j]])
        wcoef = wcoef.at[j, 1].set(wavelet[j, POW[j + 1]])
    cexp = _prep(dst3, zeros_in)
    hist1 = _run(src3, dst3, cexp, xp, zeros_in)
    seeds = _seed_call(hist1, wcoef)
    hb0, hb1 = _run2(src3, dst3, cexp, seeds[0], seeds[1], zeros_in)
    hb2 = _run(src3, dst3, cexp, seeds[2], zeros_in)
    out = _fin_call(x, hist1, [hb0, hb1, hb2], wcoef, lin_W,
                    lin_b.reshape(1, C))
    return out, wavelet


# speedup vs baseline: 1.5561x; 1.1641x over previous
import functools

import jax
import jax.numpy as jnp
from jax import lax
from jax.experimental import pallas as pl
from jax.experimental.pallas import tpu as pltpu
from jax.experimental.pallas import tpu_sc as plsc

N = 10000
NP = 10240
E = 160000
C = 128
NC = 2
NS = 16
EPT = E // NS
ECH = 50
NECH = EPT // ECH
W = 8
NW = NECH // W
NQ = NECH // 4
RPT = NP // NS
RCH = 16
NRCH = RPT // RCH
POW = (1, 2, 4, 8, 16)
STEPS = 16

_mesh = plsc.VectorSubcoreMesh(
    core_axis_name="c", subcore_axis_name="s", num_cores=NC, num_subcores=NS
)


def _fill(ref, rows, val):
    def body(i, carry):
        r = i // (C // 16)
        g = (i % (C // 16)) * 16
        ref[r, pl.ds(g, 16)] = jnp.full((16,), val, jnp.float32)
        return carry
    lax.fori_loop(0, rows * (C // 16), body, 0)


def _prep_body(dst3, zeros_in, cexp_out, deg_sp, dstb, ones, dbuf, cbuf):
    c = lax.axis_index("c")
    s = lax.axis_index("s")

    @pl.when(c == 0)
    def _():
        base = s * RPT
        _fill(ones, ECH, 1.0)
        for k in range(0, NRCH, 4):
            pltpu.sync_copy(zeros_in, deg_sp.at[pl.ds(base + k * RCH, RCH)])
            pltpu.sync_copy(
                zeros_in, deg_sp.at[pl.ds(base + (k + 1) * RCH, RCH)])
            pltpu.sync_copy(
                zeros_in, deg_sp.at[pl.ds(base + (k + 2) * RCH, RCH)])
            pltpu.sync_copy(
                zeros_in, deg_sp.at[pl.ds(base + (k + 3) * RCH, RCH)])
        pltpu.sync_copy(dst3.at[s], dstb)
        plsc.subcore_barrier()

        def ebody(j, carry):
            pltpu.sync_copy(ones, deg_sp.at[dstb.at[j]], add=True)
            return carry
        lax.fori_loop(0, NECH, ebody, 0)
        plsc.subcore_barrier()

        for k in range(0, NRCH, 4):
            pltpu.sync_copy(
                deg_sp.at[pl.ds(base + k * RCH, 4 * RCH)], dbuf)

            def cbody(i, carry):
                r = i // (C // 16)
                g = (i % (C // 16)) * 16
                dv = dbuf[r, pl.ds(g, 16)]
                cbuf[r, pl.ds(g, 16)] = jnp.where(dv > 0.0, 0.5 / dv, 0.0)
                return carry
            lax.fori_loop(0, 4 * RCH * (C // 16), cbody, 0)
            pltpu.sync_copy(cbuf, cexp_out.at[pl.ds(base + k * RCH, 4 * RCH)])


_prep = functools.partial(
    pl.kernel,
    _prep_body,
    out_type=jax.ShapeDtypeStruct((NP, C), jnp.float32),
    mesh=_mesh,
    scratch_types=[
        pltpu.VMEM_SHARED((NP, C), jnp.float32),
        pltpu.VMEM((NECH, ECH), jnp.int32),
        pltpu.VMEM((ECH, C), jnp.float32),
        pltpu.VMEM((4 * RCH, C), jnp.float32),
        pltpu.VMEM((4 * RCH, C), jnp.float32),
    ],
)()


def _diffuse(src3, dst3, cexp, seed, zeros_in, hist,
             agg_sp, srcw, dstw, rows, ubufs,
             gsems, ssems, semw, s):
    base = s * RPT
    (hbA, abA, cbA) = ubufs

    for k in range(NRCH):
        off = base + k * RCH
        pltpu.sync_copy(seed.at[pl.ds(off, RCH)], hbA)
        pltpu.sync_copy(hbA, hist.at[0, pl.ds(off, RCH)])
        pltpu.sync_copy(zeros_in, agg_sp.at[pl.ds(off, RCH)])
    plsc.subcore_barrier()

    def winload(wi, slot, sync):
        sc = pltpu.sync_copy if sync else (
            lambda a, b: pltpu.async_copy(a, b, semw))
        sc(src3.at[s, pl.ds(wi * W, W)], srcw.at[slot])
        sc(dst3.at[s, pl.ds(wi * W, W)], dstw.at[slot])

    def winwait():
        pltpu.make_async_copy(
            src3.at[s, pl.ds(0, W)], srcw.at[0], semw).wait()
        pltpu.make_async_copy(
            dst3.at[s, pl.ds(0, W)], dstw.at[0], semw).wait()

    def step(t, carry):
        hprev = hist.at[t - 1]
        winload(0, 0, True)
        winload(1, 1, False)

        def fire(j, rbuf, sem):
            wi = j // W
            wl = j - wi * W
            pltpu.async_copy(hprev.at[srcw.at[wi % 2, wl]], rbuf, sem)

        def gwait(j, rbuf, sem):
            wi = j // W
            wl = j - wi * W
            pltpu.make_async_copy(
                hprev.at[srcw.at[wi % 2, wl]], rbuf, sem).wait()

        def sfire(j, rbuf, ssem):
            wi = j // W
            wl = j - wi * W
            return pltpu.async_copy(
                rbuf, agg_sp.at[dstw.at[wi % 2, wl]], ssem, add=True)

        def winmgmt(j):
            wi = j // W
            wl = j - wi * W
            @pl.when((wl == 0) & (wi >= 1) & (wi < NW - 1))
            def _():
                winload(wi + 1, (wi + 1) % 2, False)

            @pl.when((wl == 4) & (wi < NW - 1))
            def _():
                winwait()

        for u in range(4):
            fire(u, rows[u], gsems[u])

        def quad(q, icarry):
            j0 = q * 4
            cps = []
            for u in range(4):
                gwait(j0 + u, rows[u], gsems[u])
                cps.append(sfire(j0 + u, rows[u], ssems[u]))
            for u in range(4):
                winmgmt(j0 + u)
            for u in range(4):
                cps[u].wait()
                fire(j0 + u + 4, rows[u], gsems[u])
            return icarry
        lax.fori_loop(0, NQ - 1, quad, 0)
        jt = (NQ - 1) * 4
        tail_cps = []
        for u in range(4):
            gwait(jt + u, rows[u], gsems[u])
            tail_cps.append(sfire(jt + u, rows[u], ssems[u]))
        for u in range(4):
            tail_cps[u].wait()
        plsc.subcore_barrier()

        def ucompute(hb, ab, cb):
            def ubody(i, icarry):
                r = i // (C // 16)
                g = (i % (C // 16)) * 16
                hv = hb[r, pl.ds(g, 16)]
                av = ab[r, pl.ds(g, 16)]
                cv = cb[r, pl.ds(g, 16)]
                hb[r, pl.ds(g, 16)] = 0.5 * hv + cv * av
                return icarry
            lax.fori_loop(0, RCH * (C // 16), ubody, 0)

        def ustore(k, hb, wsem):
            off = base + k * RCH
            pltpu.async_copy(hb, hist.at[t, pl.ds(off, RCH)], wsem)
            pltpu.async_copy(zeros_in, agg_sp.at[pl.ds(off, RCH)], wsem)

        def uswait(k, hb, wsem):
            off = base + k * RCH
            pltpu.make_async_copy(
                hb, hist.at[t, pl.ds(off, RCH)], wsem).wait()
            pltpu.make_async_copy(
                zeros_in, agg_sp.at[pl.ds(off, RCH)], wsem).wait()

        def uchunk(k, icarry):
            off = base + k * RCH
            ld = [
                pltpu.async_copy(agg_sp.at[pl.ds(off, RCH)], abA, gsems[0]),
                pltpu.async_copy(
                    hist.at[t - 1, pl.ds(off, RCH)], hbA, gsems[1]),
                pltpu.async_copy(cexp.at[pl.ds(off, RCH)], cbA, gsems[2]),
            ]
            for cp in ld:
                cp.wait()
            ucompute(hbA, abA, cbA)
            st = [
                pltpu.async_copy(hbA, hist.at[t, pl.ds(off, RCH)], gsems[0]),
                pltpu.async_copy(
                    zeros_in, agg_sp.at[pl.ds(off, RCH)], gsems[1]),
            ]
            for cp in st:
                cp.wait()
            return icarry
        lax.fori_loop(0, NRCH, uchunk, 0)
        plsc.subcore_barrier()
        return carry
    lax.fori_loop(1, STEPS + 1, step, 0)


def _unpack_scratch(scr):
    agg_sp = scr[0]
    srcw, dstw = scr[1], scr[2]
    rows = scr[3:7]
    ubufs = scr[7:10]
    gsems = scr[10:14]
    ssems = scr[14:18]
    semw = scr[18]
    return agg_sp, srcw, dstw, rows, ubufs, gsems, ssems, semw


def _run_body(src3, dst3, cexp, seed, zeros_in, hist, *scr):
    c = lax.axis_index("c")
    s = lax.axis_index("s")

    @pl.when(c == 0)
    def _():
        _diffuse(src3, dst3, cexp, seed, zeros_in, hist,
                 *_unpack_scratch(scr), s)


def _run2_body(src3, dst3, cexp, seed_a, seed_b, zeros_in, hist_a, hist_b,
               *scr):
    c = lax.axis_index("c")
    s = lax.axis_index("s")

    @pl.when(c == 0)
    def _():
        _diffuse(src3, dst3, cexp, seed_a, zeros_in, hist_a,
                 *_unpack_scratch(scr), s)

    @pl.when(c == 1)
    def _():
        _diffuse(src3, dst3, cexp, seed_b, zeros_in, hist_b,
                 *_unpack_scratch(scr), s)


_SC_SCRATCH = (
    [pltpu.VMEM_SHARED((NP, C), jnp.float32)]
    + [pltpu.VMEM((2, W, ECH), jnp.int32)] * 2
    + [pltpu.VMEM((ECH, C), jnp.float32)] * 4
    + [pltpu.VMEM((RCH, C), jnp.float32)] * 3
    + [pltpu.SemaphoreType.DMA] * 9
)

_run2 = functools.partial(
    pl.kernel,
    _run2_body,
    out_type=(jax.ShapeDtypeStruct((STEPS + 1, NP, C), jnp.float32),
              jax.ShapeDtypeStruct((STEPS + 1, NP, C), jnp.float32)),
    mesh=_mesh,
    scratch_types=_SC_SCRATCH,
)()

_run = functools.partial(
    pl.kernel,
    _run_body,
    out_type=jax.ShapeDtypeStruct((STEPS + 1, NP, C), jnp.float32),
    mesh=_mesh,
    scratch_types=_SC_SCRATCH,
)()


_SEED_BN = 1024


def _seed_body(h1, h2, h4, h8, wc, o0, o1, o2):
    hs = {1: h1, 2: h2, 4: h4, 8: h8}
    for j, out in enumerate((o0, o1, o2)):
        a, b = POW[j], POW[j + 1]
        out[...] = jnp.abs(hs[a][0] * wc[j:j + 1, 0:1]
                           + hs[b][0] * wc[j:j + 1, 1:2])


def _seed_call(hist1, wcoef):
    bspec = [
        pl.BlockSpec((1, _SEED_BN, C), lambda ii, t=t: (t, ii, 0))
        for t in (1, 2, 4, 8)
    ]
    bspec.append(pl.BlockSpec((8, 128), lambda ii: (0, 0)))
    out_spec = pl.BlockSpec((_SEED_BN, C), lambda ii: (ii, 0))
    return pl.pallas_call(
        _seed_body,
        grid=(NP // _SEED_BN,),
        in_specs=bspec,
        out_specs=[out_spec] * 3,
        out_shape=[jax.ShapeDtypeStruct((NP, C), jnp.float32)] * 3,
    )(hist1, hist1, hist1, hist1, wcoef)


_FIN_BN = 400
_H1_SLOTS = (1, 2, 4, 8, 16)
_HB_SLOTS = ((2, 4, 8, 16), (4, 8, 16), (8, 16))


def _fin_body(*refs):
    x_r = refs[0]
    nh1 = len(_H1_SLOTS)
    h1refs = refs[1:1 + nh1]
    pos = 1 + nh1
    hbrefs = []
    for slots in _HB_SLOTS:
        hbrefs.append(refs[pos:pos + len(slots)])
        pos += len(slots)
    wc, Wm, bb, out = refs[pos], refs[pos + 1], refs[pos + 2], refs[pos + 3]

    h1 = {t: r[0] for t, r in zip(_H1_SLOTS, h1refs)}
    hb = [{t: r[0] for t, r in zip(slots, rs)}
          for slots, rs in zip(_HB_SLOTS, hbrefs)]

    def wav(j, ha, hbv):
        return jnp.abs(ha * wc[j:j + 1, 0:1] + hbv * wc[j:j + 1, 1:2])

    feats = [x_r[...]]
    for j in range(4):
        feats.append(wav(j, h1[POW[j]], h1[POW[j + 1]]))
    for j in range(3):
        for jp in range(j + 1, 4):
            feats.append(wav(jp, hb[j][POW[jp]], hb[j][POW[jp + 1]]))
    f = jnp.concatenate(feats, axis=-1)
    f = jnp.where(f >= 0.0, f, 0.01 * f)
    acc = lax.dot_general(f, Wm[...], (((1,), (1,)), ((), ())),
                          preferred_element_type=jnp.float32)
    out[...] = acc + bb[...]


def _fin_call(x, hist1, hbs, wcoef, lin_W, lin_b2):
    in_specs = [pl.BlockSpec((_FIN_BN, C), lambda i: (i, 0))]
    args = [x]
    for t in _H1_SLOTS:
        in_specs.append(
            pl.BlockSpec((1, _FIN_BN, C), lambda i, t=t: (t, i, 0)))
        args.append(hist1)
    for bi, slots in enumerate(_HB_SLOTS):
        for t in slots:
            in_specs.append(
                pl.BlockSpec((1, _FIN_BN, C), lambda i, t=t: (t, i, 0)))
            args.append(hbs[bi])
    in_specs.append(pl.BlockSpec((8, 128), lambda i: (0, 0)))
    args.append(wcoef)
    in_specs.append(pl.BlockSpec((C, 11 * C), lambda i: (0, 0)))
    args.append(lin_W)
    in_specs.append(pl.BlockSpec((1, C), lambda i: (0, 0)))
    args.append(lin_b2)
    return pl.pallas_call(
        _fin_body,
        grid=(N // _FIN_BN,),
        in_specs=in_specs,
        out_specs=pl.BlockSpec((_FIN_BN, C), lambda i: (i, 0)),
        out_shape=jax.ShapeDtypeStruct((N, C), jnp.float32),
    )(*args)


def kernel(x, edge_index, wavelet, lin_W, lin_b):
    src3 = edge_index[0].reshape(NS, NECH, ECH)
    dst3 = edge_index[1].reshape(NS, NECH, ECH)
    xp = jnp.zeros((NP, C), jnp.float32).at[:N].set(x)
    zeros_in = jnp.zeros((RCH, C), jnp.float32)
    wcoef = jnp.zeros((8, 128), jnp.float32)
    for j in range(4):
        wcoef = wcoef.at[j, 0].set(wavelet[j, POW[j]])
        wcoef = wcoef.at[j, 1].set(wavelet[j, POW[j + 1]])
    cexp = _prep(dst3, zeros_in)
    hist1 = _run(src3, dst3, cexp, xp, zeros_in)
    seeds = _seed_call(hist1, wcoef)
    hb0, hb1 = _run2(src3, dst3, cexp, seeds[0], seeds[1], zeros_in)
    hb2 = _run(src3, dst3, cexp, seeds[2], zeros_in)
    out = _fin_call(x, hist1, [hb0, hb1, hb2], wcoef, lin_W,
                    lin_b.reshape(1, C))
    return out, wavelet


# speedup vs baseline: 1.7464x; 1.1223x over previous
import functools

import jax
import jax.numpy as jnp
from jax import lax
from jax.experimental import pallas as pl
from jax.experimental.pallas import tpu as pltpu
from jax.experimental.pallas import tpu_sc as plsc

N = 10000
NP = 10240
E = 160000
C = 128
NC = 2
NS = 16
EPT = E // NS
ECH = 50
NECH = EPT // ECH
W = 8
NW = NECH // W
NQ = NECH // 4
RPT = NP // NS
RCH = 16
NRCH = RPT // RCH
POW = (1, 2, 4, 8, 16)
STEPS = 16

_mesh = plsc.VectorSubcoreMesh(
    core_axis_name="c", subcore_axis_name="s", num_cores=NC, num_subcores=NS
)


def _fill(ref, rows, val):
    def body(i, carry):
        r = i // (C // 16)
        g = (i % (C // 16)) * 16
        ref[r, pl.ds(g, 16)] = jnp.full((16,), val, jnp.float32)
        return carry
    lax.fori_loop(0, rows * (C // 16), body, 0)


def _prep_body(dst3, zeros_in, cexp_out, deg_sp, dstb, ones, dbuf, cbuf):
    c = lax.axis_index("c")
    s = lax.axis_index("s")

    @pl.when(c == 0)
    def _():
        base = s * RPT
        _fill(ones, ECH, 1.0)
        for k in range(0, NRCH, 4):
            pltpu.sync_copy(zeros_in, deg_sp.at[pl.ds(base + k * RCH, RCH)])
            pltpu.sync_copy(
                zeros_in, deg_sp.at[pl.ds(base + (k + 1) * RCH, RCH)])
            pltpu.sync_copy(
                zeros_in, deg_sp.at[pl.ds(base + (k + 2) * RCH, RCH)])
            pltpu.sync_copy(
                zeros_in, deg_sp.at[pl.ds(base + (k + 3) * RCH, RCH)])
        pltpu.sync_copy(dst3.at[s], dstb)
        plsc.subcore_barrier()

        def ebody(j, carry):
            pltpu.sync_copy(ones, deg_sp.at[dstb.at[j]], add=True)
            return carry
        lax.fori_loop(0, NECH, ebody, 0)
        plsc.subcore_barrier()

        for k in range(0, NRCH, 4):
            pltpu.sync_copy(
                deg_sp.at[pl.ds(base + k * RCH, 4 * RCH)], dbuf)

            def cbody(i, carry):
                r = i // (C // 16)
                g = (i % (C // 16)) * 16
                dv = dbuf[r, pl.ds(g, 16)]
                cbuf[r, pl.ds(g, 16)] = jnp.where(dv > 0.0, 0.5 / dv, 0.0)
                return carry
            lax.fori_loop(0, 4 * RCH * (C // 16), cbody, 0)
            pltpu.sync_copy(cbuf, cexp_out.at[pl.ds(base + k * RCH, 4 * RCH)])


_prep = functools.partial(
    pl.kernel,
    _prep_body,
    out_type=jax.ShapeDtypeStruct((NP, C), jnp.float32),
    mesh=_mesh,
    scratch_types=[
        pltpu.VMEM_SHARED((NP, C), jnp.float32),
        pltpu.VMEM((NECH, ECH), jnp.int32),
        pltpu.VMEM((ECH, C), jnp.float32),
        pltpu.VMEM((4 * RCH, C), jnp.float32),
        pltpu.VMEM((4 * RCH, C), jnp.float32),
    ],
)()


def _diffuse(src3, dst3, cexp, seed, zeros_in, hist,
             agg_sp, srcw, dstw, rows, ubufs,
             gsems, ssems, semw, s):
    base = s * RPT
    (hbA, abA, cbA, hbB, abB, cbB) = ubufs

    for k in range(NRCH):
        off = base + k * RCH
        pltpu.sync_copy(seed.at[pl.ds(off, RCH)], hbA)
        pltpu.sync_copy(hbA, hist.at[0, pl.ds(off, RCH)])
        pltpu.sync_copy(zeros_in, agg_sp.at[pl.ds(off, RCH)])
    plsc.subcore_barrier()

    def winload(wi, slot, sync):
        sc = pltpu.sync_copy if sync else (
            lambda a, b: pltpu.async_copy(a, b, semw))
        sc(src3.at[s, pl.ds(wi * W, W)], srcw.at[slot])
        sc(dst3.at[s, pl.ds(wi * W, W)], dstw.at[slot])

    def winwait():
        pltpu.make_async_copy(
            src3.at[s, pl.ds(0, W)], srcw.at[0], semw).wait()
        pltpu.make_async_copy(
            dst3.at[s, pl.ds(0, W)], dstw.at[0], semw).wait()

    def step(t, carry):
        hprev = hist.at[t - 1]
        winload(0, 0, True)
        winload(1, 1, False)

        def fire(j, rbuf, sem):
            wi = j // W
            wl = j - wi * W
            pltpu.async_copy(hprev.at[srcw.at[wi % 2, wl]], rbuf, sem)

        def gwait(j, rbuf, sem):
            wi = j // W
            wl = j - wi * W
            pltpu.make_async_copy(
                hprev.at[srcw.at[wi % 2, wl]], rbuf, sem).wait()

        def sfire(j, rbuf, ssem):
            wi = j // W
            wl = j - wi * W
            return pltpu.async_copy(
                rbuf, agg_sp.at[dstw.at[wi % 2, wl]], ssem, add=True)

        def winmgmt(j):
            wi = j // W
            wl = j - wi * W
            @pl.when((wl == 0) & (wi >= 1) & (wi < NW - 1))
            def _():
                winload(wi + 1, (wi + 1) % 2, False)

            @pl.when((wl == 4) & (wi < NW - 1))
            def _():
                winwait()

        for u in range(4):
            fire(u, rows[u], gsems[u])

        def quad(q, icarry):
            j0 = q * 4
            cps = []
            for u in range(4):
                gwait(j0 + u, rows[u], gsems[u])
                cps.append(sfire(j0 + u, rows[u], ssems[u]))
            for u in range(4):
                winmgmt(j0 + u)
            for u in range(4):
                cps[u].wait()
                fire(j0 + u + 4, rows[u], gsems[u])
            return icarry
        lax.fori_loop(0, NQ - 1, quad, 0)
        jt = (NQ - 1) * 4
        tail_cps = []
        for u in range(4):
            gwait(jt + u, rows[u], gsems[u])
            tail_cps.append(sfire(jt + u, rows[u], ssems[u]))
        for u in range(4):
            tail_cps[u].wait()
        plsc.subcore_barrier()

        def ucompute(hb, ab, cb):
            def ubody(i, icarry):
                r = i // (C // 16)
                g = (i % (C // 16)) * 16
                hv = hb[r, pl.ds(g, 16)]
                av = ab[r, pl.ds(g, 16)]
                cv = cb[r, pl.ds(g, 16)]
                hb[r, pl.ds(g, 16)] = 0.5 * hv + cv * av
                return icarry
            lax.fori_loop(0, RCH * (C // 16), ubody, 0)

        def ustore(k, hb, wsem):
            off = base + k * RCH
            pltpu.async_copy(hb, hist.at[t, pl.ds(off, RCH)], wsem)
            pltpu.async_copy(zeros_in, agg_sp.at[pl.ds(off, RCH)], wsem)

        def uswait(k, hb, wsem):
            off = base + k * RCH
            pltpu.make_async_copy(
                hb, hist.at[t, pl.ds(off, RCH)], wsem).wait()
            pltpu.make_async_copy(
                zeros_in, agg_sp.at[pl.ds(off, RCH)], wsem).wait()

        def uload(off, hb, ab, cb, u0):
            return [
                pltpu.async_copy(agg_sp.at[pl.ds(off, RCH)], ab, gsems[u0]),
                pltpu.async_copy(
                    hist.at[t - 1, pl.ds(off, RCH)], hb, gsems[u0 + 1]),
                pltpu.async_copy(cexp.at[pl.ds(off, RCH)], cb, ssems[u0]),
            ]

        def ustore(off, hb, u0):
            return [
                pltpu.async_copy(hb, hist.at[t, pl.ds(off, RCH)],
                                 ssems[u0 + 1]),
                pltpu.async_copy(zeros_in, agg_sp.at[pl.ds(off, RCH)],
                                 gsems[u0]),
            ]

        def upair(m, icarry):
            offa = base + (2 * m) * RCH
            offb = offa + RCH
            lda = uload(offa, hbA, abA, cbA, 0)
            ldb = uload(offb, hbB, abB, cbB, 2)
            for cp in lda:
                cp.wait()
            ucompute(hbA, abA, cbA)
            sta = ustore(offa, hbA, 0)
            for cp in ldb:
                cp.wait()
            ucompute(hbB, abB, cbB)
            stb = ustore(offb, hbB, 2)
            for cp in sta + stb:
                cp.wait()
            return icarry
        lax.fori_loop(0, NRCH // 2, upair, 0)
        plsc.subcore_barrier()
        return carry
    lax.fori_loop(1, STEPS + 1, step, 0)


def _unpack_scratch(scr):
    agg_sp = scr[0]
    srcw, dstw = scr[1], scr[2]
    rows = scr[3:7]
    ubufs = scr[7:13]
    gsems = scr[13:17]
    ssems = scr[17:21]
    semw = scr[21]
    return agg_sp, srcw, dstw, rows, ubufs, gsems, ssems, semw


def _run_body(src3, dst3, cexp, seed, zeros_in, hist, *scr):
    c = lax.axis_index("c")
    s = lax.axis_index("s")

    @pl.when(c == 0)
    def _():
        _diffuse(src3, dst3, cexp, seed, zeros_in, hist,
                 *_unpack_scratch(scr), s)


def _run2_body(src3, dst3, cexp, seed_a, seed_b, zeros_in, hist_a, hist_b,
               *scr):
    c = lax.axis_index("c")
    s = lax.axis_index("s")

    @pl.when(c == 0)
    def _():
        _diffuse(src3, dst3, cexp, seed_a, zeros_in, hist_a,
                 *_unpack_scratch(scr), s)

    @pl.when(c == 1)
    def _():
        _diffuse(src3, dst3, cexp, seed_b, zeros_in, hist_b,
                 *_unpack_scratch(scr), s)


_SC_SCRATCH = (
    [pltpu.VMEM_SHARED((NP, C), jnp.float32)]
    + [pltpu.VMEM((2, W, ECH), jnp.int32)] * 2
    + [pltpu.VMEM((ECH, C), jnp.float32)] * 4
    + [pltpu.VMEM((RCH, C), jnp.float32)] * 6
    + [pltpu.SemaphoreType.DMA] * 9
)

_run2 = functools.partial(
    pl.kernel,
    _run2_body,
    out_type=(jax.ShapeDtypeStruct((STEPS + 1, NP, C), jnp.float32),
              jax.ShapeDtypeStruct((STEPS + 1, NP, C), jnp.float32)),
    mesh=_mesh,
    scratch_types=_SC_SCRATCH,
)()

_run = functools.partial(
    pl.kernel,
    _run_body,
    out_type=jax.ShapeDtypeStruct((STEPS + 1, NP, C), jnp.float32),
    mesh=_mesh,
    scratch_types=_SC_SCRATCH,
)()


_SEED_BN = 1024


def _seed_body(h1, h2, h4, h8, wc, o0, o1, o2):
    hs = {1: h1, 2: h2, 4: h4, 8: h8}
    for j, out in enumerate((o0, o1, o2)):
        a, b = POW[j], POW[j + 1]
        out[...] = jnp.abs(hs[a][0] * wc[j:j + 1, 0:1]
                           + hs[b][0] * wc[j:j + 1, 1:2])


def _seed_call(hist1, wcoef):
    bspec = [
        pl.BlockSpec((1, _SEED_BN, C), lambda ii, t=t: (t, ii, 0))
        for t in (1, 2, 4, 8)
    ]
    bspec.append(pl.BlockSpec((8, 128), lambda ii: (0, 0)))
    out_spec = pl.BlockSpec((_SEED_BN, C), lambda ii: (ii, 0))
    return pl.pallas_call(
        _seed_body,
        grid=(NP // _SEED_BN,),
        in_specs=bspec,
        out_specs=[out_spec] * 3,
        out_shape=[jax.ShapeDtypeStruct((NP, C), jnp.float32)] * 3,
    )(hist1, hist1, hist1, hist1, wcoef)


_FIN_BN = 400
_H1_SLOTS = (1, 2, 4, 8, 16)
_HB_SLOTS = ((2, 4, 8, 16), (4, 8, 16), (8, 16))


def _fin_body(*refs):
    x_r = refs[0]
    nh1 = len(_H1_SLOTS)
    h1refs = refs[1:1 + nh1]
    pos = 1 + nh1
    hbrefs = []
    for slots in _HB_SLOTS:
        hbrefs.append(refs[pos:pos + len(slots)])
        pos += len(slots)
    wc, Wm, bb, out = refs[pos], refs[pos + 1], refs[pos + 2], refs[pos + 3]

    h1 = {t: r[0] for t, r in zip(_H1_SLOTS, h1refs)}
    hb = [{t: r[0] for t, r in zip(slots, rs)}
          for slots, rs in zip(_HB_SLOTS, hbrefs)]

    def wav(j, ha, hbv):
        return jnp.abs(ha * wc[j:j + 1, 0:1] + hbv * wc[j:j + 1, 1:2])

    feats = [x_r[...]]
    for j in range(4):
        feats.append(wav(j, h1[POW[j]], h1[POW[j + 1]]))
    for j in range(3):
        for jp in range(j + 1, 4):
            feats.append(wav(jp, hb[j][POW[jp]], hb[j][POW[jp + 1]]))
    f = jnp.concatenate(feats, axis=-1)
    f = jnp.where(f >= 0.0, f, 0.01 * f)
    acc = lax.dot_general(f, Wm[...], (((1,), (1,)), ((), ())),
                          preferred_element_type=jnp.float32)
    out[...] = acc + bb[...]


def _fin_call(x, hist1, hbs, wcoef, lin_W, lin_b2):
    in_specs = [pl.BlockSpec((_FIN_BN, C), lambda i: (i, 0))]
    args = [x]
    for t in _H1_SLOTS:
        in_specs.append(
            pl.BlockSpec((1, _FIN_BN, C), lambda i, t=t: (t, i, 0)))
        args.append(hist1)
    for bi, slots in enumerate(_HB_SLOTS):
        for t in slots:
            in_specs.append(
                pl.BlockSpec((1, _FIN_BN, C), lambda i, t=t: (t, i, 0)))
            args.append(hbs[bi])
    in_specs.append(pl.BlockSpec((8, 128), lambda i: (0, 0)))
    args.append(wcoef)
    in_specs.append(pl.BlockSpec((C, 11 * C), lambda i: (0, 0)))
    args.append(lin_W)
    in_specs.append(pl.BlockSpec((1, C), lambda i: (0, 0)))
    args.append(lin_b2)
    return pl.pallas_call(
        _fin_body,
        grid=(N // _FIN_BN,),
        in_specs=in_specs,
        out_specs=pl.BlockSpec((_FIN_BN, C), lambda i: (i, 0)),
        out_shape=jax.ShapeDtypeStruct((N, C), jnp.float32),
    )(*args)


def kernel(x, edge_index, wavelet, lin_W, lin_b):
    src3 = edge_index[0].reshape(NS, NECH, ECH)
    dst3 = edge_index[1].reshape(NS, NECH, ECH)
    xp = jnp.zeros((NP, C), jnp.float32).at[:N].set(x)
    zeros_in = jnp.zeros((RCH, C), jnp.float32)
    wcoef = jnp.zeros((8, 128), jnp.float32)
    for j in range(4):
        wcoef = wcoef.at[j, 0].set(wavelet[j, POW[j]])
        wcoef = wcoef.at[j, 1].set(wavelet[j, POW[j + 1]])
    cexp = _prep(dst3, zeros_in)
    hist1 = _run(src3, dst3, cexp, xp, zeros_in)
    seeds = _seed_call(hist1, wcoef)
    hb0, hb1 = _run2(src3, dst3, cexp, seeds[0], seeds[1], zeros_in)
    hb2 = _run(src3, dst3, cexp, seeds[2], zeros_in)
    out = _fin_call(x, hist1, [hb0, hb1, hb2], wcoef, lin_W,
                    lin_b.reshape(1, C))
    return out, wavelet
